# baseline (speedup 1.0000x reference)
import functools

import jax
import jax.numpy as jnp
from jax.experimental import pallas as pl
from jax.experimental.pallas import tpu as pltpu

ALPHA = 0.2
PJ = 512
ROWS1 = 256
ROWS2 = 256
GROUPS = ROWS1 // 8
SLABS = 8


def _row_dot(ab, wh):
    return jax.lax.dot_general(
        ab, wh, (((0,), (1,)), ((), ())),
        preferred_element_type=jnp.float32)


def _pack_matrix():
    g = jax.lax.broadcasted_iota(jnp.int32, (GROUPS, ROWS1), 0)
    i = jax.lax.broadcasted_iota(jnp.int32, (GROUPS, ROWS1), 1)
    w = jnp.where(i % GROUPS == g, jnp.left_shift(1, i // GROUPS), 0)
    return w.astype(jnp.bfloat16)


def _finish(acc, csum, n, nh):
    s = acc[:, nh:nh + 1]
    mean_wh = csum / n
    hp = jnp.where(s > 0.0, acc[:, :nh] / jnp.where(s > 0.0, s, 1.0), mean_wh)
    return jnp.where(hp > 0.0, hp, jnp.exp(hp) - 1.0)


def _mega_body(x_ref, adj_ref, w1_ref, a1t_ref, a1b_ref, w2_ref, a2t_ref,
               a2b_ref, wl_ref, bl_ref, out_ref,
               whe1_s, f1a_s, f2t_s, csum1_s, pk_s, whe2_s, f1b_s, f2bt_s,
               csum2_s, pr1_s, pr2_s, *, n, nh, a_steps, b_steps):
    i = pl.program_id(0)

    pj = x_ref.shape[0]

    @pl.when(i < a_steps)
    def _proj_phase():
        wh = jnp.dot(x_ref[...], w1_ref[...],
                     preferred_element_type=jnp.float32)
        base = i * pj
        whe1_s[pl.ds(base, pj), :nh] = wh.astype(jnp.bfloat16)
        whe1_s[pl.ds(base, pj), nh:] = jnp.ones((pj, 1), jnp.bfloat16)
        f1a_s[pl.ds(base, pj), :] = jnp.dot(
            wh, a1t_ref[...], preferred_element_type=jnp.float32)
        f2t_s[0:1, pl.ds(base, pj)] = _row_dot(a1b_ref[...], wh)

        @pl.when(i == 0)
        def _():
            csum1_s[...] = jnp.zeros_like(csum1_s)

        rowid = base + jax.lax.broadcasted_iota(jnp.int32, (pj, 1), 0)
        csum1_s[...] += jnp.sum(jnp.where(rowid < n, wh, 0.0),
                                axis=0, keepdims=True)

    @pl.when(jnp.logical_and(i >= a_steps, i < a_steps + b_steps))
    def _attn1_phase():
        b = i - a_steps

        @pl.when(b == 0)
        def _():
            f2row = f2t_s[0:1, :n]
            m2 = jnp.max(f2row)
            pr1_s[0:1, :] = jnp.exp(f2row - m2)
            pr1_s[1:2, :] = jnp.exp(ALPHA * f2row - m2)
            csum2_s[...] = jnp.zeros_like(csum2_s)

        p_col = pr1_s[0:1, :]
        r_col = pr1_s[1:2, :]
        f1 = f1a_s[pl.ds(b * ROWS1, ROWS1), :]
        q_row = jnp.exp((ALPHA - 1.0) * f1)
        valid = jnp.where(adj_ref[...] > 0.0, 1.0, 0.0)
        pmf = jnp.maximum(p_col, q_row * r_col) * valid
        pm = pmf.astype(jnp.bfloat16)

        packed = jnp.dot(_pack_matrix(), valid.astype(jnp.bfloat16),
                         preferred_element_type=jnp.float32)
        pk_s[pl.ds(b * GROUPS, GROUPS), :] = (
            packed.astype(jnp.int32).astype(jnp.uint8))

        acc = jnp.dot(pm, whe1_s[pl.ds(0, n), :],
                      preferred_element_type=jnp.float32)
        h1 = _finish(acc, csum1_s[...], n, nh)

        wh2 = jnp.dot(h1, w2_ref[...], preferred_element_type=jnp.float32)
        base = b * ROWS1
        whe2_s[pl.ds(base, ROWS1), :nh] = wh2.astype(jnp.bfloat16)
        whe2_s[pl.ds(base, ROWS1), nh:] = jnp.ones((ROWS1, 1), jnp.bfloat16)
        f1b_s[pl.ds(base, ROWS1), :] = jnp.dot(
            wh2, a2t_ref[...], preferred_element_type=jnp.float32)
        f2bt_s[0:1, pl.ds(base, ROWS1)] = _row_dot(a2b_ref[...], wh2)

        rowid = base + jax.lax.broadcasted_iota(
            jnp.int32, (ROWS1, 1), 0)
        csum2_s[...] += jnp.sum(jnp.where(rowid < n, wh2, 0.0),
                                axis=0, keepdims=True)

    @pl.when(i >= a_steps + b_steps)
    def _attn2_phase():
        c = i - a_steps - b_steps

        @pl.when(c == 0)
        def _():
            f2row = f2bt_s[0:1, :n]
            m2 = jnp.max(f2row)
            pr2_s[0:1, :] = jnp.exp(f2row - m2).astype(jnp.bfloat16)
            pr2_s[1:2, :] = jnp.exp(ALPHA * f2row - m2).astype(jnp.bfloat16)

        p_col = pr2_s[0:1, :]
        r_col = pr2_s[1:2, :]
        f1 = f1b_s[pl.ds(c * ROWS2, ROWS2), :]
        q_row = jnp.exp((ALPHA - 1.0) * f1).astype(jnp.bfloat16)

        pk = pk_s[pl.ds(c * (ROWS2 // 8), ROWS2 // 8), :].astype(jnp.int32)
        slabs = []
        for ch in range(ROWS2 // ROWS1):
            chunk = pk[ch * GROUPS:(ch + 1) * GROUPS, :]
            for r in range(SLABS):
                bits = jnp.bitwise_and(jnp.right_shift(chunk, r), 1)
                slabs.append(bits.astype(jnp.bfloat16))
        m = jnp.concatenate(slabs, axis=0)

        pm = jnp.maximum(p_col, q_row * r_col) * m
        acc = jnp.dot(pm, whe2_s[pl.ds(0, n), :],
                      preferred_element_type=jnp.float32)
        h2 = _finish(acc, csum2_s[...], n, nh)
        out_ref[...] = (
            jnp.dot(h2, wl_ref[...], preferred_element_type=jnp.float32)
            + bl_ref[...]
        )


def kernel(x, adj, W1, a1, W2, a2, Wl, bl):
    n, nfeat = x.shape
    nh = W1.shape[1]
    no = Wl.shape[1]
    pj = min(PJ, n)
    a_steps = pl.cdiv(n, pj)
    b_steps = pl.cdiv(n, ROWS1)
    c_steps = pl.cdiv(n, ROWS2)
    grid = a_steps + b_steps + c_steps
    npad1 = b_steps * ROWS1
    npad_a = a_steps * pj
    npad_f1 = max(npad_a, npad1)

    ab = a_steps + b_steps
    const = lambda i: (0, 0)

    out = pl.pallas_call(
        functools.partial(_mega_body, n=n, nh=nh,
                          a_steps=a_steps, b_steps=b_steps),
        grid=(grid,),
        in_specs=[
            pl.BlockSpec((pj, nfeat),
                         lambda i: (jnp.minimum(i, a_steps - 1), 0)),
            pl.BlockSpec((ROWS1, n),
                         lambda i: (jnp.clip(i - a_steps, 0, b_steps - 1),
                                    0)),
            pl.BlockSpec((nfeat, nh), const),
            pl.BlockSpec((nh, 1), const),
            pl.BlockSpec((nh, 1), const),
            pl.BlockSpec((nh, nh), const),
            pl.BlockSpec((nh, 1), const),
            pl.BlockSpec((nh, 1), const),
            pl.BlockSpec((nh, no), const),
            pl.BlockSpec((1, no), const),
        ],
        out_specs=pl.BlockSpec(
            (ROWS2, no), lambda i: (jnp.clip(i - ab, 0, c_steps - 1), 0)),
        out_shape=jax.ShapeDtypeStruct((n, no), jnp.float32),
        scratch_shapes=[
            pltpu.VMEM((npad_a, nh + 1), jnp.bfloat16),
            pltpu.VMEM((npad_f1, 1), jnp.float32),
            pltpu.VMEM((1, npad_a), jnp.float32),
            pltpu.VMEM((1, nh), jnp.float32),
            pltpu.VMEM((b_steps * GROUPS, n), jnp.uint8),
            pltpu.VMEM((npad1, nh + 1), jnp.bfloat16),
            pltpu.VMEM((npad1, 1), jnp.float32),
            pltpu.VMEM((1, npad1), jnp.float32),
            pltpu.VMEM((1, nh), jnp.float32),
            pltpu.VMEM((8, n), jnp.float32),
            pltpu.VMEM((8, n), jnp.bfloat16),
        ],
        compiler_params=pltpu.CompilerParams(
            dimension_semantics=("arbitrary",)),
    )(x, adj, W1, a1[:nh], a1[nh:], W2, a2[:nh], a2[nh:], Wl,
      bl.reshape(1, -1))

    return out


# speedup vs baseline: 1.0747x; 1.0747x over previous
import functools

import jax
import jax.numpy as jnp
from jax.experimental import pallas as pl
from jax.experimental.pallas import tpu as pltpu

ALPHA = 0.2
ROWS1 = 256
ROWS2 = 512
GROUPS = 32
SLABS = 8


def _proj_math(h, w, at, ab):
    wh = jnp.dot(h, w, preferred_element_type=jnp.float32)
    f1 = jnp.dot(wh, at, preferred_element_type=jnp.float32)
    f2 = jnp.dot(wh, ab, preferred_element_type=jnp.float32)
    return wh, f1, f2


def _proj_body(h_ref, w_ref, at_ref, ab_ref, whe_ref, f1_ref, f2_ref,
               csum_ref):
    wh, f1, f2 = _proj_math(h_ref[...], w_ref[...], at_ref[...], ab_ref[...])
    nh = wh.shape[1]
    whe_ref[:, :nh] = wh.astype(jnp.bfloat16)
    whe_ref[:, nh:] = jnp.ones((wh.shape[0], 1), jnp.bfloat16)
    f1_ref[...] = f1
    f2_ref[...] = f2

    @pl.when(pl.program_id(0) == 0)
    def _():
        csum_ref[...] = jnp.zeros_like(csum_ref)

    csum_ref[...] += jnp.sum(wh, axis=0, keepdims=True)


def _project(h, w, a, rows_blk):
    n = h.shape[0]
    nh = w.shape[1]
    at, ab = a[:nh], a[nh:]
    grid = n // rows_blk
    return pl.pallas_call(
        _proj_body,
        grid=(grid,),
        in_specs=[
            pl.BlockSpec((rows_blk, h.shape[1]), lambda i: (i, 0)),
            pl.BlockSpec((w.shape[0], nh), lambda i: (0, 0)),
            pl.BlockSpec((nh, 1), lambda i: (0, 0)),
            pl.BlockSpec((nh, 1), lambda i: (0, 0)),
        ],
        out_specs=[
            pl.BlockSpec((rows_blk, nh + 1), lambda i: (i, 0)),
            pl.BlockSpec((rows_blk, 1), lambda i: (i, 0)),
            pl.BlockSpec((rows_blk, 1), lambda i: (i, 0)),
            pl.BlockSpec((1, nh), lambda i: (0, 0)),
        ],
        out_shape=[
            jax.ShapeDtypeStruct((n, nh + 1), jnp.bfloat16),
            jax.ShapeDtypeStruct((n, 1), jnp.float32),
            jax.ShapeDtypeStruct((n, 1), jnp.float32),
            jax.ShapeDtypeStruct((1, nh), jnp.float32),
        ],
    )(h, w, at, ab)


def _finish(acc, csum, n, nh):
    s = acc[:, nh:nh + 1]
    mean_wh = csum / n
    hp = jnp.where(s > 0.0, acc[:, :nh] / jnp.where(s > 0.0, s, 1.0), mean_wh)
    return jnp.where(hp > 0.0, hp, jnp.exp(hp) - 1.0)


def _pack_matrix():
    g = jax.lax.broadcasted_iota(jnp.int32, (GROUPS, ROWS1), 0)
    i = jax.lax.broadcasted_iota(jnp.int32, (GROUPS, ROWS1), 1)
    w = jnp.where(i % GROUPS == g, jnp.left_shift(1, i // GROUPS), 0)
    return w.astype(jnp.bfloat16)


def _attn1_body(adj_ref, f1_ref, f2t_ref, whe_ref, csum_ref, w2_ref,
                a2t_ref, a2b_ref, whe2_ref, f1b_ref, f2b_ref, csum2_ref,
                pk_ref, pr_ref, *, nh, n):
    @pl.when(pl.program_id(0) == 0)
    def _():
        f2row = f2t_ref[...]
        m2 = jnp.max(f2row)
        pr_ref[0:1, :] = jnp.exp(f2row - m2)
        pr_ref[1:2, :] = jnp.exp(ALPHA * f2row - m2)

    p_col = pr_ref[0:1, :]
    r_col = pr_ref[1:2, :]
    q_row = jnp.exp((ALPHA - 1.0) * f1_ref[...])
    valid = jnp.where(adj_ref[...] > 0.0, 1.0, 0.0)
    pmf = jnp.maximum(p_col, q_row * r_col) * valid
    pm = pmf.astype(jnp.bfloat16)

    packed = jnp.dot(_pack_matrix(), valid.astype(jnp.bfloat16),
                     preferred_element_type=jnp.float32)
    pk_ref[...] = packed.astype(jnp.int32).astype(jnp.uint8)

    acc = jnp.dot(pm, whe_ref[...], preferred_element_type=jnp.float32)
    h1 = _finish(acc, csum_ref[...], n, nh)

    wh2, f1b, f2b = _proj_math(h1, w2_ref[...], a2t_ref[...], a2b_ref[...])
    whe2_ref[:, :nh] = wh2.astype(jnp.bfloat16)
    whe2_ref[:, nh:] = jnp.ones((wh2.shape[0], 1), jnp.bfloat16)
    f1b_ref[...] = f1b
    f2b_ref[...] = f2b

    rowid = (pl.program_id(0) * ROWS1
             + jax.lax.broadcasted_iota(jnp.int32, (wh2.shape[0], 1), 0))
    wh2m = jnp.where(rowid < n, wh2, 0.0)

    @pl.when(pl.program_id(0) == 0)
    def _():
        csum2_ref[...] = jnp.zeros_like(csum2_ref)

    csum2_ref[...] += jnp.sum(wh2m, axis=0, keepdims=True)


def _attn2_body(pk_ref, f1_ref, f2t_ref, whe_ref, csum_ref, wl_ref, bl_ref,
                out_ref, pr_ref, *, nh, n):
    @pl.when(pl.program_id(0) == 0)
    def _():
        f2row = f2t_ref[...]
        m2 = jnp.max(f2row)
        pr_ref[0:1, :] = jnp.exp(f2row - m2).astype(jnp.bfloat16)
        pr_ref[1:2, :] = jnp.exp(ALPHA * f2row - m2).astype(jnp.bfloat16)

    p_col = pr_ref[0:1, :]
    r_col = pr_ref[1:2, :]
    q_row = jnp.exp((ALPHA - 1.0) * f1_ref[...]).astype(jnp.bfloat16)

    pk = pk_ref[...]
    slabs = []
    for c in range(ROWS2 // ROWS1):
        chunk = pk[c * GROUPS:(c + 1) * GROUPS, :]
        for r in range(SLABS):
            bit = jnp.bitwise_and(chunk, jnp.uint8(1 << r))
            slabs.append(bit.astype(jnp.bfloat16) * jnp.bfloat16(2.0 ** -r))
    m = jnp.concatenate(slabs, axis=0)

    pm = jnp.maximum(p_col, q_row * r_col) * m
    acc = jnp.dot(pm, whe_ref[...], preferred_element_type=jnp.float32)
    h2 = _finish(acc, csum_ref[...], n, nh)
    out_ref[...] = (
        jnp.dot(h2, wl_ref[...], preferred_element_type=jnp.float32)
        + bl_ref[...]
    )


def kernel(x, adj, W1, a1, W2, a2, Wl, bl):
    n, nfeat = x.shape
    nh = W1.shape[1]
    grid1 = pl.cdiv(n, ROWS1)
    grid2 = pl.cdiv(n, ROWS2)
    n_groups = grid1 * GROUPS
    pj_rows = 1000 if n % 1000 == 0 else n

    whe1, f1a, f2a, csum1 = _project(x, W1, a1, pj_rows)
    f2t = f2a.reshape(1, n)

    colv_spec = pl.BlockSpec((1, n), lambda i: (0, 0))
    whe_spec = pl.BlockSpec((n, nh + 1), lambda i: (0, 0))
    csum_spec = pl.BlockSpec((1, nh), lambda i: (0, 0))

    whe2, f1b, f2b, csum2, packed = pl.pallas_call(
        functools.partial(_attn1_body, nh=nh, n=n),
        grid=(grid1,),
        in_specs=[
            pl.BlockSpec((ROWS1, n), lambda i: (i, 0)),
            pl.BlockSpec((ROWS1, 1), lambda i: (i, 0)),
            colv_spec, whe_spec, csum_spec,
            pl.BlockSpec((nh, nh), lambda i: (0, 0)),
            pl.BlockSpec((nh, 1), lambda i: (0, 0)),
            pl.BlockSpec((nh, 1), lambda i: (0, 0)),
        ],
        out_specs=[
            pl.BlockSpec((ROWS1, nh + 1), lambda i: (i, 0)),
            pl.BlockSpec((ROWS1, 1), lambda i: (i, 0)),
            pl.BlockSpec((ROWS1, 1), lambda i: (i, 0)),
            csum_spec,
            pl.BlockSpec((GROUPS, n), lambda i: (i, 0)),
        ],
        out_shape=[
            jax.ShapeDtypeStruct((n, nh + 1), jnp.bfloat16),
            jax.ShapeDtypeStruct((n, 1), jnp.float32),
            jax.ShapeDtypeStruct((n, 1), jnp.float32),
            jax.ShapeDtypeStruct((1, nh), jnp.float32),
            jax.ShapeDtypeStruct((n_groups, n), jnp.uint8),
        ],
        scratch_shapes=[pltpu.VMEM((8, n), jnp.float32)],
        compiler_params=pltpu.CompilerParams(
            dimension_semantics=("arbitrary",)),
    )(adj, f1a, f2t, whe1, csum1, W2, a2[:nh], a2[nh:])

    f2tb = f2b.reshape(1, n)

    out = pl.pallas_call(
        functools.partial(_attn2_body, nh=nh, n=n),
        grid=(grid2,),
        in_specs=[
            pl.BlockSpec((ROWS2 // SLABS, n), lambda i: (i, 0)),
            pl.BlockSpec((ROWS2, 1), lambda i: (i, 0)),
            colv_spec, whe_spec, csum_spec,
            pl.BlockSpec((nh, Wl.shape[1]), lambda i: (0, 0)),
            pl.BlockSpec((1, Wl.shape[1]), lambda i: (0, 0)),
        ],
        out_specs=pl.BlockSpec((ROWS2, Wl.shape[1]), lambda i: (i, 0)),
        out_shape=jax.ShapeDtypeStruct((n, Wl.shape[1]), jnp.float32),
        scratch_shapes=[pltpu.VMEM((8, n), jnp.bfloat16)],
        compiler_params=pltpu.CompilerParams(
            dimension_semantics=("arbitrary",)),
    )(packed, f1b, f2tb, whe2, csum2, Wl, bl.reshape(1, -1))

    return out
